# baseline (speedup 1.0000x reference)
import functools

import jax
import jax.numpy as jnp
from jax.experimental import pallas as pl
from jax.experimental.pallas import tpu as pltpu

_INTER = 512
_E = 8


def _moe_body(x_ref, gu_hbm, dn_hbm, idx_ref, wts_ref, out_ref,
              gu_buf, dn_buf, gu_sem, gu_sem2, dn_sem):
    half = _INTER

    def gu_copy_a(e):
        return pltpu.make_async_copy(
            gu_hbm.at[e, :half], gu_buf.at[e % 2, :half], gu_sem.at[e % 2])

    def gu_copy_b(e):
        return pltpu.make_async_copy(
            gu_hbm.at[e, half:], gu_buf.at[e % 2, half:], gu_sem2.at[e % 2])

    def dn_copy(e):
        return pltpu.make_async_copy(
            dn_hbm.at[e], dn_buf.at[e % 2], dn_sem.at[e % 2])

    gu_copy_a(0).start()
    gu_copy_b(0).start()
    dn_copy(0).start()
    x = x_ref[...]
    for e in range(_E):
        if e + 1 < _E:
            gu_copy_a(e + 1).start()
            gu_copy_b(e + 1).start()
            dn_copy(e + 1).start()
        gu_copy_a(e).wait()
        gu_copy_b(e).wait()
        proj = jax.lax.dot_general(
            x, gu_buf[e % 2], (((1,), (1,)), ((), ())),
            preferred_element_type=jnp.float32)
        gate = proj[:, :_INTER]
        up = proj[:, _INTER:]
        h = gate * jax.nn.sigmoid(gate) * up
        dn_copy(e).wait()
        out_e = jax.lax.dot_general(
            h, dn_buf[e % 2], (((1,), (1,)), ((), ())),
            preferred_element_type=jnp.float32)
        sel = (idx_ref[...] == e).astype(jnp.float32)
        w = jnp.sum(wts_ref[...] * sel, axis=1, keepdims=True)
        contrib = out_e * w
        if e == 0:
            out_ref[...] = contrib
        else:
            out_ref[...] += contrib


@jax.jit
def kernel(hidden_states, top_k_index, top_k_weights, gate_up_proj, down_proj):
    n, h = hidden_states.shape
    e = gate_up_proj.shape[0]
    i2 = gate_up_proj.shape[1]
    i = down_proj.shape[2]
    out = pl.pallas_call(
        _moe_body,
        in_specs=[
            pl.BlockSpec(memory_space=pltpu.MemorySpace.VMEM),
            pl.BlockSpec(memory_space=pltpu.MemorySpace.HBM),
            pl.BlockSpec(memory_space=pltpu.MemorySpace.HBM),
            pl.BlockSpec(memory_space=pltpu.MemorySpace.VMEM),
            pl.BlockSpec(memory_space=pltpu.MemorySpace.VMEM),
        ],
        out_specs=pl.BlockSpec(memory_space=pltpu.MemorySpace.VMEM),
        out_shape=jax.ShapeDtypeStruct((n, h), jnp.float32),
        scratch_shapes=[
            pltpu.VMEM((2, i2, h), jnp.float32),
            pltpu.VMEM((2, h, i), jnp.float32),
            pltpu.SemaphoreType.DMA((2,)),
            pltpu.SemaphoreType.DMA((2,)),
            pltpu.SemaphoreType.DMA((2,)),
        ],
    )(hidden_states, gate_up_proj, down_proj,
      top_k_index.astype(jnp.int32), top_k_weights)
    return out.astype(hidden_states.dtype)
